# baseline (speedup 1.0000x reference)
import functools

import jax
import jax.numpy as jnp
from jax import lax
from jax.experimental import pallas as pl
from jax.experimental.pallas import tpu as pltpu
from jax.experimental.pallas import tpu_sc as plsc

S = 224
B = 4
NV = 6
N = 32768
NC, NS = 2, 16
C = 2048
NG = C // 16
NCH = N // C
FBW = S * S
FBP = FBW + 512
DUMMY_PK = (FBW + 32) | (3 << 16) | (3 << 18)

SEGMENTS = {
    0: [(0, 0, 12, "keep")],
    1: [(0, 12, 16, "ship"), (1, 0, 8, "ship")],
    2: [(2, 0, 4, "ship"), (1, 8, 16, "keep")],
    3: [(2, 4, 16, "keep")],
}
MM_SRC = {0: ((0, 0), (1, 0)), 1: ((1, 1), (2, 1)), 2: ((2, 0), (3, 0))}
OWNER = {0: 0, 1: 2, 2: 3}


def _splat_body(pts_ref, tbl_ref, out_ref, ship_hbm,
                xa, ya, za, xb, yb, zb, pkc, ftc, fb0, fb1, cvec, tmp16,
                mmx_sh, sema, semb):
    cid = lax.axis_index("c")
    sid = lax.axis_index("s")
    gid = sid // 4
    role = sid - gid * 4

    lane = lax.iota(jnp.int32, 16)
    nine = lane < 9
    uvec = jnp.where(nine, lane % 3, 0)
    wvec = jnp.where(nine, lane // 3, 0)
    rvec = uvec + wvec * S
    zerov = jnp.zeros((16,), jnp.float32)
    inf = jnp.float32(jnp.inf)

    def job_of(j):
        return cid * 12 + gid * 3 + j

    def load_consts(j):
        v = job_of(j) % NV
        pltpu.sync_copy(tbl_ref.at[pl.ds(v * 128, 128)], cvec)
        return (cvec[pl.ds(0, 16)], cvec[pl.ds(16, 16)],
                cvec[pl.ds(32, 16)], cvec[pl.ds(48, 16)],
                cvec[pl.ds(64, 16)], cvec[pl.ds(80, 16)])

    bufs_a = (xa, ya, za)
    bufs_b = (xb, yb, zb)

    def _issue(j, ch, bufs, sem):
        base_in = (job_of(j) // NV) * 3 * N + ch * C
        for q, d in enumerate(bufs):
            pltpu.async_copy(pts_ref.at[pl.ds(base_in + q * N, C)], d, sem)

    def _wait(j, ch, bufs, sem):
        base_in = (job_of(j) // NV) * 3 * N + ch * C
        for q, d in enumerate(bufs):
            pltpu.make_async_copy(
                pts_ref.at[pl.ds(base_in + q * N, C)], d, sem).wait()

    def phase_a(role_segments):
        for seg_idx, (j, lo, hi, _act) in enumerate(role_segments):
            ca, sa, ce, se, dmin, dmax = load_consts(j)
            hc = (hi - lo) // 2

            def _mm_chunk(bufs, mn, mx, ca=ca, sa=sa, ce=ce, se=se):
                x_ref, y_ref, z_ref = bufs

                def _grp(g, c2):
                    mn2, mx2 = c2
                    sl = pl.ds(g * 16, 16)
                    zf = (y_ref[sl] * se
                          + (x_ref[sl] * sa + z_ref[sl] * ca) * ce)
                    return jnp.minimum(mn2, zf), jnp.maximum(mx2, zf)

                return lax.fori_loop(0, NG, _grp, (mn, mx))

            _issue(j, lo, bufs_a, sema)

            def _mm_pair(cp, carry, j=j, lo=lo, hc=hc, _mm_chunk=_mm_chunk):
                mn, mx = carry
                _issue(j, lo + 2 * cp + 1, bufs_b, semb)
                _wait(j, lo + 2 * cp, bufs_a, sema)
                mn, mx = _mm_chunk(bufs_a, mn, mx)

                @pl.when(cp < hc - 1)
                def _():
                    _issue(j, lo + 2 * cp + 2, bufs_a, sema)

                _wait(j, lo + 2 * cp + 1, bufs_b, semb)
                return _mm_chunk(bufs_b, mn, mx)

            mn, mx = lax.fori_loop(
                0, hc, _mm_pair,
                (jnp.full((16,), inf, jnp.float32),
                 jnp.full((16,), -inf, jnp.float32)))
            slot = (sid * 2 + seg_idx) * 32
            tmp16[...] = mn
            pltpu.sync_copy(tmp16, mmx_sh.at[pl.ds(slot, 16)])
            tmp16[...] = mx
            pltpu.sync_copy(tmp16, mmx_sh.at[pl.ds(slot + 16, 16)])

    for r in range(4):
        @pl.when(role == r)
        def _(r=r):
            phase_a(SEGMENTS[r])

    plsc.subcore_barrier()

    def _lane_all(vec, op):
        cur = vec
        for k in (1, 2, 4, 8):
            tmp16[...] = cur
            cur = op(cur, plsc.load_gather(tmp16, [lane ^ k]))
        return cur

    def job_minmax(j):
        (s0, e0), (s1, e1) = MM_SRC[j]
        base0 = ((gid * 4 + s0) * 2 + e0) * 32
        base1 = ((gid * 4 + s1) * 2 + e1) * 32
        pltpu.sync_copy(mmx_sh.at[pl.ds(base0, 32)], cvec.at[pl.ds(0, 32)])
        pltpu.sync_copy(mmx_sh.at[pl.ds(base1, 32)], cvec.at[pl.ds(32, 32)])
        mn = jnp.minimum(cvec[pl.ds(0, 16)], cvec[pl.ds(32, 16)])
        mx = jnp.maximum(cvec[pl.ds(16, 16)], cvec[pl.ds(48, 16)])
        zmin = _lane_all(mn, jnp.minimum)
        zmax = _lane_all(mx, jnp.maximum)
        return zmin, (zmax - zmin) + 1e-6

    def zero_fbs():
        def _zrow(r2, _):
            for q in range(4):
                fb0[pl.ds(r2 * 64 + q * 16, 16)] = zerov
                fb1[pl.ds(r2 * 64 + q * 16, 16)] = zerov
            return 0
        lax.fori_loop(0, FBW // 64, _zrow, 0)

    def merge_fbs():
        def _mrow(r2, _):
            for q in range(4):
                sl = pl.ds(r2 * 64 + q * 16, 16)
                fb0[sl] = jnp.maximum(fb0[sl], fb1[sl])
            return 0
        lax.fori_loop(0, FBW // 64, _mrow, 0)

    def _splat_chunk(bufs, consts, zmin, den):
        ca, sa, ce, se, dmin, dmax = consts
        x_ref, y_ref, z_ref = bufs

        def _grp(g, cnt):
            sl = pl.ds(g * 16, 16)
            x = x_ref[sl]
            y = y_ref[sl]
            z = z_ref[sl]
            x_rot = x * ca - z * sa
            z_rot = x * sa + z * ca
            y_rot = y * ce - z_rot * se
            zf = y * se + z_rot * ce
            ft = 0.3 + 0.7 * ((zf - zmin) / den)

            def _pf(base, d):
                return ((base + d) + 1.0) * 0.5 * (S - 1)
            fxl = _pf(x_rot, dmin)
            fxh = _pf(x_rot, dmax)
            fyl = _pf(y_rot, dmin)
            fyh = _pf(y_rot, dmax)
            keep = (fxh > -1.0) & (fxl < 224.0) & (fyh > -1.0) & (fyl < 224.0)

            def _cl(f):
                return jnp.minimum(jnp.maximum(f, 0.0), 223.0).astype(jnp.int32)
            lo_x = _cl(fxl)
            hi_x = _cl(fxh)
            lo_y = _cl(fyl)
            hi_y = _cl(fyh)
            pk = ((lo_y * S + lo_x)
                  | ((hi_x - lo_x) << 16)
                  | ((hi_y - lo_y) << 18))
            plsc.store_compressed(pkc.at[pl.ds(cnt, 16)], pk, mask=keep)
            plsc.store_compressed(ftc.at[pl.ds(cnt, 16)], ft, mask=keep)
            inc = plsc.all_reduce_population_count(keep)
            return cnt + lax.squeeze(lax.slice(inc, (0,), (1,)), (0,))

        cnt = lax.fori_loop(0, NG, _grp, 0)
        pkc[pl.ds(cnt, 16)] = jnp.full((16,), DUMMY_PK, jnp.int32)

        def _rmw(g, _):
            base = g * 16
            pkv = pkc[pl.ds(base, 16)]
            ftv = ftc[pl.ds(base, 16)]
            for i in range(16):
                iv = jnp.full((16,), i, jnp.int32)
                pk = jnp.take_along_axis(pkv, iv, axis=0)
                ft = jnp.take_along_axis(ftv, iv, axis=0)
                ok = (uvec <= ((pk >> 16) & 3)) & (wvec <= (pk >> 18))
                idxf = (pk & 0xFFFF) + rvec
                f = fb0 if i % 2 == 0 else fb1
                cur = plsc.load_gather(f, [idxf], mask=ok)
                plsc.store_scatter(f, [idxf], jnp.maximum(cur, ft), mask=ok)
            return 0

        lax.fori_loop(0, (cnt + 15) // 16, _rmw, 0)

    def phase_b(role_segments):
        for j, lo, hi, act in role_segments:
            consts = load_consts(j)
            zmin, den = job_minmax(j)
            zero_fbs()
            hc = (hi - lo) // 2
            _issue(j, lo, bufs_a, sema)

            def _sp_pair(cp, _, j=j, lo=lo, hc=hc, consts=consts,
                         zmin=zmin, den=den):
                _issue(j, lo + 2 * cp + 1, bufs_b, semb)
                _wait(j, lo + 2 * cp, bufs_a, sema)
                _splat_chunk(bufs_a, consts, zmin, den)

                @pl.when(cp < hc - 1)
                def _():
                    _issue(j, lo + 2 * cp + 2, bufs_a, sema)

                _wait(j, lo + 2 * cp + 1, bufs_b, semb)
                _splat_chunk(bufs_b, consts, zmin, den)
                return 0

            lax.fori_loop(0, hc, _sp_pair, 0)
            merge_fbs()
            if act == "ship":
                ship_base = (cid * 12 + gid * 3 + j) * FBW
                pltpu.sync_copy(fb0.at[pl.ds(0, FBW)],
                                ship_hbm.at[pl.ds(ship_base, FBW)])

    for r in range(4):
        @pl.when(role == r)
        def _(r=r):
            phase_b(SEGMENTS[r])

    plsc.subcore_barrier()

    def phase_c(j):
        ship_base = (cid * 12 + gid * 3 + j) * FBW

        def _piece(p, _):
            pltpu.sync_copy(ship_hbm.at[pl.ds(ship_base + p * C, C)], xa)

            def _m(g, _2):
                sl = pl.ds(g * 16, 16)
                dsl = pl.ds(p * C + g * 16, 16)
                fb0[dsl] = jnp.maximum(fb0[dsl], xa[sl])
                return 0
            lax.fori_loop(0, NG, _m, 0)
            return 0
        lax.fori_loop(0, FBW // C, _piece, 0)
        rem = FBW - (FBW // C) * C
        if rem:
            pltpu.sync_copy(
                ship_hbm.at[pl.ds(ship_base + (FBW // C) * C, rem)],
                xa.at[pl.ds(0, rem)])

            def _mr(g, _2):
                sl = pl.ds(g * 16, 16)
                dsl = pl.ds((FBW // C) * C + g * 16, 16)
                fb0[dsl] = jnp.maximum(fb0[dsl], xa[sl])
                return 0
            lax.fori_loop(0, rem // 16, _mr, 0)
        out_base = job_of(j) * FBW
        pltpu.sync_copy(fb0.at[pl.ds(0, FBW)],
                        out_ref.at[pl.ds(out_base, FBW)])

    for jj, r in OWNER.items():
        @pl.when(role == r)
        def _(jj=jj):
            phase_c(jj)


@jax.jit
def kernel(points):
    az = jnp.linspace(0.0, 360.0, NV + 1)[:-1]
    el = jnp.array([0.0, 30.0, -30.0, 0.0, 0.0, 0.0])[:NV]
    azr = az * jnp.pi / 180.0
    elr = el * jnp.pi / 180.0
    offs = jnp.linspace(-2.0 / S, 2.0 / S, 5)
    dmin = jnp.full((NV,), offs[0])
    dmax = jnp.full((NV,), offs[4])
    zero = jnp.zeros((NV,))
    tbl = jnp.stack(
        [jnp.cos(azr), jnp.sin(azr), jnp.cos(elr), jnp.sin(elr),
         dmin, dmax, zero, zero], axis=1)
    tbl16 = jnp.broadcast_to(tbl[:, :, None], (NV, 8, 16))
    tbl_flat = tbl16.astype(jnp.float32).reshape(-1)
    pts_flat = points.transpose(0, 2, 1).reshape(-1)

    mesh = plsc.VectorSubcoreMesh(core_axis_name="c", subcore_axis_name="s")
    run = functools.partial(
        pl.kernel,
        mesh=mesh,
        compiler_params=pltpu.CompilerParams(needs_layout_passes=False),
        out_type=(jax.ShapeDtypeStruct((B * NV * FBW,), jnp.float32),
                  jax.ShapeDtypeStruct((NC * 12 * FBW,), jnp.float32)),
        scratch_types=[
            pltpu.VMEM((C,), jnp.float32),
            pltpu.VMEM((C,), jnp.float32),
            pltpu.VMEM((C,), jnp.float32),
            pltpu.VMEM((C,), jnp.float32),
            pltpu.VMEM((C,), jnp.float32),
            pltpu.VMEM((C,), jnp.float32),
            pltpu.VMEM((C + 16,), jnp.int32),
            pltpu.VMEM((C + 16,), jnp.float32),
            pltpu.VMEM((FBP,), jnp.float32),
            pltpu.VMEM((FBP,), jnp.float32),
            pltpu.VMEM((128,), jnp.float32),
            pltpu.VMEM((16,), jnp.float32),
            pltpu.VMEM_SHARED((NS * 2 * 32,), jnp.float32),
            pltpu.SemaphoreType.DMA,
            pltpu.SemaphoreType.DMA,
        ],
    )(_splat_body)
    img, _ship = run(pts_flat, tbl_flat)
    img = img.reshape(B, NV, 1, S, S)
    return jnp.broadcast_to(img, (B, NV, 3, S, S))


# speedup vs baseline: 1.5107x; 1.5107x over previous
import functools

import jax
import jax.numpy as jnp
from jax import lax
from jax.experimental import pallas as pl
from jax.experimental.pallas import tpu as pltpu
from jax.experimental.pallas import tpu_sc as plsc

S = 224
B = 4
NV = 6
N = 32768
NJOBS = B * NV
NC, NS = 2, 16
C = 2048
NG = C // 16
NCH = N // C
HC = NCH // 2
FBW = S * S
FBP = FBW + 512
DUMMY_PK = (FBW + 32) | (3 << 16) | (3 << 18)


def _splat_body(pts_ref, tbl_ref, out_ref,
                xa, ya, za, xb, yb, zb, pkc, ftc, fb0, fb1, cvec, tmp16,
                sema, semb):
    wid = lax.axis_index("s") * NC + lax.axis_index("c")

    @pl.when(wid < NJOBS)
    def _job():
        b = wid // NV
        v = wid - b * NV

        pltpu.sync_copy(tbl_ref.at[pl.ds(v * 128, 128)], cvec)
        ca = cvec[pl.ds(0, 16)]
        sa = cvec[pl.ds(16, 16)]
        ce = cvec[pl.ds(32, 16)]
        se = cvec[pl.ds(48, 16)]
        dmin = cvec[pl.ds(64, 16)]
        dmax = cvec[pl.ds(80, 16)]

        zerov = jnp.zeros((16,), jnp.float32)

        def _zrow(r, _):
            for q in range(4):
                fb0[pl.ds(r * 64 + q * 16, 16)] = zerov
                fb1[pl.ds(r * 64 + q * 16, 16)] = zerov
            return 0
        lax.fori_loop(0, FBW // 64, _zrow, 0)

        bufs_a = (xa, ya, za)
        bufs_b = (xb, yb, zb)

        def _issue(ch, bufs, sem):
            base_in = b * 3 * N + ch * C
            for j, d in enumerate(bufs):
                pltpu.async_copy(pts_ref.at[pl.ds(base_in + j * N, C)], d, sem)

        def _wait(ch, bufs, sem):
            base_in = b * 3 * N + ch * C
            for j, d in enumerate(bufs):
                pltpu.make_async_copy(
                    pts_ref.at[pl.ds(base_in + j * N, C)], d, sem).wait()

        def _mm_chunk(bufs, mn, mx):
            x_ref, y_ref, z_ref = bufs

            def _grp(g, c2):
                mn2, mx2 = c2
                sl = pl.ds(g * 16, 16)
                x = x_ref[sl]
                y = y_ref[sl]
                z = z_ref[sl]
                z_rot = x * sa + z * ca
                zf = y * se + z_rot * ce
                return jnp.minimum(mn2, zf), jnp.maximum(mx2, zf)

            return lax.fori_loop(0, NG, _grp, (mn, mx))

        _issue(0, bufs_a, sema)
        inf = jnp.float32(jnp.inf)

        def _mm_pair(cp, carry):
            mn, mx = carry
            _issue(2 * cp + 1, bufs_b, semb)
            _wait(2 * cp, bufs_a, sema)
            mn, mx = _mm_chunk(bufs_a, mn, mx)

            @pl.when(cp < HC - 1)
            def _():
                _issue(2 * cp + 2, bufs_a, sema)

            _wait(2 * cp + 1, bufs_b, semb)
            return _mm_chunk(bufs_b, mn, mx)

        mn, mx = lax.fori_loop(
            0, HC, _mm_pair,
            (jnp.full((16,), inf, jnp.float32), jnp.full((16,), -inf, jnp.float32)))

        lane = lax.iota(jnp.int32, 16)

        def _lane_all(vec, op):
            cur = vec
            for k in (1, 2, 4, 8):
                tmp16[...] = cur
                cur = op(cur, plsc.load_gather(tmp16, [lane ^ k]))
            return cur

        zmin = _lane_all(mn, jnp.minimum)
        zmax = _lane_all(mx, jnp.maximum)
        den = zmax - zmin + 1e-6

        nine = lane < 9
        uvec = jnp.where(nine, lane % 3, 3)
        wvec = jnp.where(nine, lane // 3, 0)
        rvec = uvec + wvec * S

        def _splat_chunk(bufs):
            x_ref, y_ref, z_ref = bufs

            def _grp(g, cnt):
                sl = pl.ds(g * 16, 16)
                x = x_ref[sl]
                y = y_ref[sl]
                z = z_ref[sl]
                x_rot = x * ca - z * sa
                z_rot = x * sa + z * ca
                y_rot = y * ce - z_rot * se
                zf = y * se + z_rot * ce
                ft = 0.3 + 0.7 * ((zf - zmin) / den)

                def _pf(base, d):
                    return ((base + d) + 1.0) * 0.5 * (S - 1)
                fxl = _pf(x_rot, dmin)
                fxh = _pf(x_rot, dmax)
                fyl = _pf(y_rot, dmin)
                fyh = _pf(y_rot, dmax)
                keep = (fxh > -1.0) & (fxl < 224.0) & (fyh > -1.0) & (fyl < 224.0)

                def _cl(f):
                    return jnp.minimum(jnp.maximum(f, 0.0), 223.0).astype(jnp.int32)
                lo_x = _cl(fxl)
                hi_x = _cl(fxh)
                lo_y = _cl(fyl)
                hi_y = _cl(fyh)
                pk = ((lo_y * S + lo_x)
                      | ((hi_x - lo_x) << 16)
                      | ((hi_y - lo_y) << 18))
                plsc.store_compressed(pkc.at[pl.ds(cnt, 16)], pk, mask=keep)
                plsc.store_compressed(ftc.at[pl.ds(cnt, 16)], ft, mask=keep)
                inc = plsc.all_reduce_population_count(keep)
                return cnt + lax.squeeze(lax.slice(inc, (0,), (1,)), (0,))

            cnt = lax.fori_loop(0, NG, _grp, 0)
            pkc[pl.ds(cnt, 16)] = jnp.full((16,), DUMMY_PK, jnp.int32)

            def _rmw(g, _):
                base = g * 16
                pkv = pkc[pl.ds(base, 16)]
                ftv = ftc[pl.ds(base, 16)]
                for i in range(16):
                    iv = jnp.full((16,), i, jnp.int32)
                    pk = jnp.take_along_axis(pkv, iv, axis=0)
                    ft = jnp.take_along_axis(ftv, iv, axis=0)
                    ok = (uvec <= ((pk >> 16) & 3)) & (wvec <= (pk >> 18))
                    idxf = (pk & 0xFFFF) + rvec
                    f = fb0 if i % 2 == 0 else fb1
                    cur = plsc.load_gather(f, [idxf], mask=ok)
                    plsc.store_scatter(f, [idxf], jnp.maximum(cur, ft), mask=ok)
                return 0

            lax.fori_loop(0, (cnt + 15) // 16, _rmw, 0)

        _issue(0, bufs_a, sema)

        def _sp_pair(cp, _):
            _issue(2 * cp + 1, bufs_b, semb)
            _wait(2 * cp, bufs_a, sema)
            _splat_chunk(bufs_a)

            @pl.when(cp < HC - 1)
            def _():
                _issue(2 * cp + 2, bufs_a, sema)

            _wait(2 * cp + 1, bufs_b, semb)
            _splat_chunk(bufs_b)
            return 0

        lax.fori_loop(0, HC, _sp_pair, 0)

        def _mrow(r, _):
            for q in range(4):
                sl = pl.ds(r * 64 + q * 16, 16)
                fb0[sl] = jnp.maximum(fb0[sl], fb1[sl])
            return 0
        lax.fori_loop(0, FBW // 64, _mrow, 0)

        out_base = (b * NV + v) * FBW
        pltpu.sync_copy(fb0.at[pl.ds(0, FBW)], out_ref.at[pl.ds(out_base, FBW)])


@jax.jit
def kernel(points):
    az = jnp.linspace(0.0, 360.0, NV + 1)[:-1]
    el = jnp.array([0.0, 30.0, -30.0, 0.0, 0.0, 0.0])[:NV]
    azr = az * jnp.pi / 180.0
    elr = el * jnp.pi / 180.0
    offs = jnp.linspace(-2.0 / S, 2.0 / S, 5)
    dmin = jnp.full((NV,), offs[0])
    dmax = jnp.full((NV,), offs[4])
    zero = jnp.zeros((NV,))
    tbl = jnp.stack(
        [jnp.cos(azr), jnp.sin(azr), jnp.cos(elr), jnp.sin(elr),
         dmin, dmax, zero, zero], axis=1)
    tbl16 = jnp.broadcast_to(tbl[:, :, None], (NV, 8, 16))
    tbl_flat = tbl16.astype(jnp.float32).reshape(-1)
    pts_flat = points.transpose(0, 2, 1).reshape(-1)

    mesh = plsc.VectorSubcoreMesh(core_axis_name="c", subcore_axis_name="s")
    run = functools.partial(
        pl.kernel,
        mesh=mesh,
        compiler_params=pltpu.CompilerParams(needs_layout_passes=False),
        out_type=jax.ShapeDtypeStruct((B * NV * FBW,), jnp.float32),
        scratch_types=[
            pltpu.VMEM((C,), jnp.float32),
            pltpu.VMEM((C,), jnp.float32),
            pltpu.VMEM((C,), jnp.float32),
            pltpu.VMEM((C,), jnp.float32),
            pltpu.VMEM((C,), jnp.float32),
            pltpu.VMEM((C,), jnp.float32),
            pltpu.VMEM((C + 16,), jnp.int32),
            pltpu.VMEM((C + 16,), jnp.float32),
            pltpu.VMEM((FBP,), jnp.float32),
            pltpu.VMEM((FBP,), jnp.float32),
            pltpu.VMEM((128,), jnp.float32),
            pltpu.VMEM((16,), jnp.float32),
            pltpu.SemaphoreType.DMA,
            pltpu.SemaphoreType.DMA,
        ],
    )(_splat_body)
    img = run(pts_flat, tbl_flat).reshape(B, NV, 1, S, S)
    return jnp.broadcast_to(img, (B, NV, 3, S, S))
